# baseline (speedup 1.0000x reference)
import functools

import jax
import jax.numpy as jnp
from jax import lax
from jax.experimental import pallas as pl
from jax.experimental.pallas import tpu as pltpu
from jax.experimental.pallas import tpu_sc as plsc

_DPAD = 128
_NBUF = 2
_G1 = 128


@functools.cache
def _make_pad(V: int, D: int):
    rows = 2048
    grid = (V + rows - 1) // rows

    def pad_block(x_ref, o_ref):
        o_ref[:, :D] = x_ref[...]
        o_ref[:, D:] = jnp.zeros_like(o_ref[:, D:])

    return pl.pallas_call(
        pad_block,
        grid=(grid,),
        in_specs=[pl.BlockSpec((rows, D), lambda i: (i, 0))],
        out_specs=pl.BlockSpec((rows, _DPAD), lambda i: (i, 0)),
        out_shape=jax.ShapeDtypeStruct((V, _DPAD), jnp.float32),
    )


@functools.cache
def _make_gather(S: int, T: int, D: int):
    info = plsc.get_sparse_core_info()
    nw = info.num_cores * info.num_subcores
    s_per_w = S // nw
    idx_per_w = s_per_w * T
    n_groups = s_per_w // _NBUF
    g2 = T - _G1
    mesh = plsc.VectorSubcoreMesh(core_axis_name="c", subcore_axis_name="s")

    @functools.partial(
        pl.kernel,
        out_type=jax.ShapeDtypeStruct((S, T, D), jnp.float32),
        mesh=mesh,
        scratch_types=[
            pltpu.VMEM((idx_per_w,), jnp.int32),
            [pltpu.VMEM((T, _DPAD), jnp.float32)] * _NBUF,
            [pltpu.VMEM((T, D), jnp.float32)] * _NBUF,
            [pltpu.SemaphoreType.DMA] * _NBUF,
            [pltpu.SemaphoreType.DMA] * _NBUF,
        ],
    )
    def gather_kernel(table_hbm, idx_hbm, out_hbm, idx_v, wide, packed,
                      gsems, wsems):
        wid = lax.axis_index("s") * info.num_cores + lax.axis_index("c")
        seq_base = wid * s_per_w

        pltpu.sync_copy(idx_hbm.at[pl.ds(wid * idx_per_w, idx_per_w)], idx_v)

        def gather_descs(s, b):
            i0 = s * T
            return (
                pltpu.make_async_copy(
                    table_hbm.at[idx_v.at[pl.ds(i0, _G1)]],
                    wide[b].at[pl.ds(0, _G1)],
                    gsems[b],
                ),
                pltpu.make_async_copy(
                    table_hbm.at[idx_v.at[pl.ds(i0 + _G1, g2)]],
                    wide[b].at[pl.ds(_G1, g2)],
                    gsems[b],
                ),
            )

        def wait_write(b):
            pltpu.make_async_copy(
                packed[b], out_hbm.at[seq_base], wsems[b]
            ).wait()

        def compact(b):
            def rows4(r4, carry):
                r = r4 * 4
                for dr in range(4):
                    for k in (0, 16, 32, 48, 64, 80, D - 16):
                        packed[b][r + dr, pl.ds(k, 16)] = (
                            wide[b][r + dr, pl.ds(k, 16)]
                        )
                return carry

            lax.fori_loop(0, T // 4, rows4, 0)

        def body(g, carry):
            s0 = g * _NBUF
            for b in range(_NBUF):
                @pl.when(g > 0)
                def _():
                    wait_write(b)
                d1, d2 = gather_descs(s0 + b, b)
                d1.start()
                d2.start()
            for b in range(_NBUF):
                d1, d2 = gather_descs(s0 + b, b)
                d1.wait()
                d2.wait()
                compact(b)
                pltpu.async_copy(
                    packed[b], out_hbm.at[seq_base + s0 + b], wsems[b]
                )
            return carry

        lax.fori_loop(0, n_groups, body, 0)
        for b in range(_NBUF):
            wait_write(b)

    return gather_kernel


def kernel(sequence, embedding_matrix):
    S, T = sequence.shape
    V, D = embedding_matrix.shape
    idx = sequence.reshape(S * T).astype(jnp.int32)
    table_p = _make_pad(V, D)(embedding_matrix)
    return _make_gather(S, T, D)(table_p, idx)
